# baseline (speedup 1.0000x reference)
import functools

import jax
import jax.numpy as jnp
from jax import lax
from jax.experimental import pallas as pl
from jax.experimental.pallas import tpu as pltpu
from jax.experimental.pallas import tpu_sc as plsc

B = 16384
E = 16
NM = 1000000
NT = 100000
NC = 2
NS = 16
NW = NC * NS
BPW = B // NW
CHW = 128
CH = BPW // CHW

GM = (NM + 127) // 128
GT = (NT + 127) // 128
MTAIL = NM - (GM - 1) * 128
TTAIL = NT - (GT - 1) * 128


def _reblock_table(tbl, tail_tbl, out, wid, n_groups, tail_wid, inflight, sem):
    nfull = n_groups - 1
    per_w = (nfull + NW - 1) // NW
    for j in range(per_w):
        g = jnp.minimum(j * NW + wid, nfull - 1)
        cp = pltpu.async_copy(
            tbl.at[:, pl.ds(pl.multiple_of(g * 128, 128), 128)], out.at[g], sem)
        inflight.append(cp)
        if len(inflight) > 16:
            inflight.pop(0).wait()

    @pl.when(wid == tail_wid)
    def _():
        pltpu.sync_copy(tail_tbl, out.at[nfull])


def _sc_reblock_body(emgT, etgT, emmT, etmT, emgQ, etgQ, emmQ, etmQ,
                     ag, tg, am, tm, sem):
    wid = lax.axis_index("s") * NC + lax.axis_index("c")
    inflight = []
    _reblock_table(emgT, emgQ, ag, wid, GM, 28, inflight, sem)
    _reblock_table(emmT, emmQ, am, wid, GM, 29, inflight, sem)
    _reblock_table(etgT, etgQ, tg, wid, GT, 30, inflight, sem)
    _reblock_table(etmT, etmQ, tm, wid, GT, 31, inflight, sem)
    for cp in inflight:
        cp.wait()


@functools.cache
def _sc_reblock():
    return pl.kernel(
        _sc_reblock_body,
        out_type=(
            jax.ShapeDtypeStruct((GM, E, 128), jnp.float32),
            jax.ShapeDtypeStruct((GT, E, 128), jnp.float32),
            jax.ShapeDtypeStruct((GM, 2 * E, 128), jnp.float32),
            jax.ShapeDtypeStruct((GT, 2 * E, 128), jnp.float32),
        ),
        mesh=plsc.VectorSubcoreMesh(core_axis_name="c", subcore_axis_name="s"),
        scratch_types=[pltpu.SemaphoreType.DMA],
        compiler_params=pltpu.CompilerParams(use_tc_tiling_on_sc=True),
    )


def _off_segments(idx_ref, off_ref, span, n_rows, tail):
    nlimit = n_rows - tail
    shift = 128 - tail
    for ch in range(CH):
        for k in range(CHW // 16):
            sl = pl.ds(k * 16, 16)
            v = idx_ref[ch, sl]
            off = (lax.shift_right_logical(v, 7) * span
                   + lax.bitwise_and(v, 127))
            off_ref[ch, sl] = jnp.where(v >= nlimit, off + shift, off)


def _sc_gather_body(mid_hbm, tid_hbm, ag, tg, am, tm,
                    emg_o, etg_o, emm_o, etm_o,
                    idx_m, idx_t, off_mg, off_mm, off_tg, off_tm,
                    emg_v, etg_v, emm_v, etm_v, sem):
    wid = lax.axis_index("s") * NC + lax.axis_index("c")
    row0 = wid * CH
    pltpu.sync_copy(mid_hbm.at[pl.ds(row0, CH)], idx_m)
    pltpu.sync_copy(tid_hbm.at[pl.ds(row0, CH)], idx_t)
    _off_segments(idx_m, off_mg, E * 128, NM, MTAIL)
    _off_segments(idx_m, off_mm, 2 * E * 128, NM, MTAIL)
    _off_segments(idx_t, off_tg, E * 128, NT, TTAIL)
    _off_segments(idx_t, off_tm, 2 * E * 128, NT, TTAIL)

    copies = []
    for ch in range(CH):
        for c in range(E):
            copies.append(pltpu.async_copy(
                ag.at[pl.ds(c * 128, GM * E * 128 - c * 128)].at[off_mg.at[ch]],
                emg_v.at[c, ch], sem))
            copies.append(pltpu.async_copy(
                tg.at[pl.ds(c * 128, GT * E * 128 - c * 128)].at[off_tg.at[ch]],
                etg_v.at[c, ch], sem))
        for c in range(2 * E):
            copies.append(pltpu.async_copy(
                am.at[pl.ds(c * 128, GM * 2 * E * 128 - c * 128)].at[off_mm.at[ch]],
                emm_v.at[c, ch], sem))
            copies.append(pltpu.async_copy(
                tm.at[pl.ds(c * 128, GT * 2 * E * 128 - c * 128)].at[off_tm.at[ch]],
                etm_v.at[c, ch], sem))
    for cp in copies:
        cp.wait()

    w0 = wid * CH
    pltpu.sync_copy(emg_v, emg_o.at[:, pl.ds(w0, CH), :])
    pltpu.sync_copy(etg_v, etg_o.at[:, pl.ds(w0, CH), :])
    pltpu.sync_copy(emm_v, emm_o.at[:, pl.ds(w0, CH), :])
    pltpu.sync_copy(etm_v, etm_o.at[:, pl.ds(w0, CH), :])


@functools.cache
def _sc_gather():
    return pl.kernel(
        _sc_gather_body,
        out_type=(
            jax.ShapeDtypeStruct((E, B // 128, 128), jnp.float32),
            jax.ShapeDtypeStruct((E, B // 128, 128), jnp.float32),
            jax.ShapeDtypeStruct((2 * E, B // 128, 128), jnp.float32),
            jax.ShapeDtypeStruct((2 * E, B // 128, 128), jnp.float32),
        ),
        mesh=plsc.VectorSubcoreMesh(core_axis_name="c", subcore_axis_name="s"),
        scratch_types=[
            pltpu.VMEM((CH, CHW), jnp.int32),
            pltpu.VMEM((CH, CHW), jnp.int32),
            pltpu.VMEM((CH, CHW), jnp.int32),
            pltpu.VMEM((CH, CHW), jnp.int32),
            pltpu.VMEM((CH, CHW), jnp.int32),
            pltpu.VMEM((CH, CHW), jnp.int32),
            pltpu.VMEM((E, CH, CHW), jnp.float32),
            pltpu.VMEM((E, CH, CHW), jnp.float32),
            pltpu.VMEM((2 * E, CH, CHW), jnp.float32),
            pltpu.VMEM((2 * E, CH, CHW), jnp.float32),
            pltpu.SemaphoreType.DMA,
        ],
        compiler_params=pltpu.CompilerParams(use_tc_tiling_on_sc=False),
    )


WIN = 16


def _tc_body(emg_ref, etg_ref, mm_ref, tt_ref, w1a, w1b, b1r, w2, b2r, w3, b3r,
             wog, wom, bor, out_ref):
    dn = (((1,), (0,)), ((), ()))
    mm = mm_ref[:]
    tt = tt_ref[:]
    h = (lax.dot_general(w1a[:], mm, dn)
         + lax.dot_general(w1b[:], tt, dn) + b1r[:].reshape(64, 1, 1))
    h = jnp.maximum(h, 0.0)
    h = jnp.maximum(lax.dot_general(w2[:], h, dn) + b2r[:].reshape(32, 1, 1), 0.0)
    h = jnp.maximum(lax.dot_general(w3[:], h, dn) + b3r[:].reshape(16, 1, 1), 0.0)
    g = emg_ref[:] * etg_ref[:]
    out = (lax.dot_general(wog[:], g, dn)[0]
           + lax.dot_general(wom[:], h, dn)[0] + bor[0, 0])
    out_ref[:] = out


def _tc_mlp(emgT, etgT, mmT, ttT, w1a, w1b, b1r, w2, b2r, w3, b3r, wog, wom, bor):
    full2 = lambda shape: pl.BlockSpec(shape, lambda i: (0, 0))
    return pl.pallas_call(
        _tc_body,
        grid=(B // 128 // WIN,),
        in_specs=[
            pl.BlockSpec((E, WIN, 128), lambda i: (0, i, 0)),
            pl.BlockSpec((E, WIN, 128), lambda i: (0, i, 0)),
            pl.BlockSpec((2 * E, WIN, 128), lambda i: (0, i, 0)),
            pl.BlockSpec((2 * E, WIN, 128), lambda i: (0, i, 0)),
            full2((64, 2 * E)),
            full2((64, 2 * E)),
            pl.BlockSpec((64,), lambda i: (0,)),
            full2((32, 64)),
            pl.BlockSpec((32,), lambda i: (0,)),
            full2((16, 32)),
            pl.BlockSpec((16,), lambda i: (0,)),
            full2((1, E)),
            full2((1, 16)),
            full2((1, 1)),
        ],
        out_specs=pl.BlockSpec((WIN, 128), lambda i: (i, 0)),
        out_shape=jax.ShapeDtypeStruct((B // 128, 128), jnp.float32),
    )(emgT, etgT, mmT, ttT, w1a, w1b, b1r, w2, b2r, w3, b3r, wog, wom, bor)


def kernel(model_ids, task_ids, Emg, Etg, Emm, Etm, W1, b1, W2, b2, W3, b3, Wo, bo):
    mid2 = model_ids.reshape(B // CHW, CHW)
    tid2 = task_ids.reshape(B // CHW, CHW)
    ag, tg, am, tm = _sc_reblock()(
        Emg.T, Etg.T, Emm.T, Etm.T,
        Emg[NM - 128:].T, Etg[NT - 128:].T, Emm[NM - 128:].T, Etm[NT - 128:].T)
    emgT, etgT, mmT, ttT = _sc_gather()(
        mid2, tid2,
        ag.reshape(-1), tg.reshape(-1), am.reshape(-1), tm.reshape(-1))
    out2 = _tc_mlp(
        emgT, etgT, mmT, ttT,
        W1[: 2 * E].T, W1[2 * E:].T, b1,
        W2.T, b2,
        W3.T, b3,
        Wo[:E].reshape(1, E), Wo[E:].reshape(1, 16),
        bo.reshape(1, 1),
    )
    return out2.reshape(B)


# speedup vs baseline: 8.9307x; 8.9307x over previous
import functools

import jax
import jax.numpy as jnp
from jax import lax
from jax.experimental import pallas as pl
from jax.experimental.pallas import tpu as pltpu
from jax.experimental.pallas import tpu_sc as plsc

B = 16384
E = 16
NM = 1000000
NT = 100000
NC = 2
NS = 16
NW = NC * NS
BPW = B // NW
CHW = 128
CH = BPW // CHW

GM = (NM + 127) // 128
GT = (NT + 127) // 128
MTAIL = NM - (GM - 1) * 128
TTAIL = NT - (GT - 1) * 128


RBK = 16


def _tc_reblock(tblT, n_groups):
    c_dim = tblT.shape[0]

    def body(in_ref, out_ref):
        x = in_ref[:]
        out_ref[:] = jnp.transpose(x.reshape(c_dim, RBK, 128), (1, 0, 2))

    return pl.pallas_call(
        body,
        grid=((n_groups + RBK - 1) // RBK,),
        in_specs=[pl.BlockSpec((c_dim, RBK * 128), lambda i: (0, i))],
        out_specs=pl.BlockSpec((RBK, c_dim, 128), lambda i: (i, 0, 0)),
        out_shape=jax.ShapeDtypeStruct((n_groups, c_dim, 128), jnp.float32),
    )(tblT)


def _off_segments(idx_ref, off_ref, span):
    for ch in range(CH):
        for k in range(CHW // 16):
            sl = pl.ds(k * 16, 16)
            v = idx_ref[ch, sl]
            off_ref[ch, sl] = (
                lax.shift_right_logical(v, 7) * span
                + lax.bitwise_and(v, 127))


def _sc_gather_body(mid_hbm, tid_hbm, ag, tg, am, tm,
                    emg_o, etg_o, emm_o, etm_o,
                    idx_m, idx_t, off_mg, off_mm, off_tg, off_tm,
                    emg_v, etg_v, emm_v, etm_v, sem):
    wid = lax.axis_index("s") * NC + lax.axis_index("c")
    row0 = wid * CH
    pltpu.sync_copy(mid_hbm.at[pl.ds(row0, CH)], idx_m)
    pltpu.sync_copy(tid_hbm.at[pl.ds(row0, CH)], idx_t)
    _off_segments(idx_m, off_mg, E * 128)
    _off_segments(idx_m, off_mm, 2 * E * 128)
    _off_segments(idx_t, off_tg, E * 128)
    _off_segments(idx_t, off_tm, 2 * E * 128)

    copies = []
    for ch in range(CH):
        for c in range(E):
            copies.append(pltpu.async_copy(
                ag.at[pl.ds(c * 128, GM * E * 128 - c * 128)].at[off_mg.at[ch]],
                emg_v.at[c, ch], sem))
            copies.append(pltpu.async_copy(
                tg.at[pl.ds(c * 128, GT * E * 128 - c * 128)].at[off_tg.at[ch]],
                etg_v.at[c, ch], sem))
        for c in range(2 * E):
            copies.append(pltpu.async_copy(
                am.at[pl.ds(c * 128, GM * 2 * E * 128 - c * 128)].at[off_mm.at[ch]],
                emm_v.at[c, ch], sem))
            copies.append(pltpu.async_copy(
                tm.at[pl.ds(c * 128, GT * 2 * E * 128 - c * 128)].at[off_tm.at[ch]],
                etm_v.at[c, ch], sem))
    for cp in copies:
        cp.wait()

    w0 = wid * CH
    pltpu.sync_copy(emg_v, emg_o.at[:, pl.ds(w0, CH), :])
    pltpu.sync_copy(etg_v, etg_o.at[:, pl.ds(w0, CH), :])
    pltpu.sync_copy(emm_v, emm_o.at[:, pl.ds(w0, CH), :])
    pltpu.sync_copy(etm_v, etm_o.at[:, pl.ds(w0, CH), :])


@functools.cache
def _sc_gather():
    return pl.kernel(
        _sc_gather_body,
        out_type=(
            jax.ShapeDtypeStruct((E, B // 128, 128), jnp.float32),
            jax.ShapeDtypeStruct((E, B // 128, 128), jnp.float32),
            jax.ShapeDtypeStruct((2 * E, B // 128, 128), jnp.float32),
            jax.ShapeDtypeStruct((2 * E, B // 128, 128), jnp.float32),
        ),
        mesh=plsc.VectorSubcoreMesh(core_axis_name="c", subcore_axis_name="s"),
        scratch_types=[
            pltpu.VMEM((CH, CHW), jnp.int32),
            pltpu.VMEM((CH, CHW), jnp.int32),
            pltpu.VMEM((CH, CHW), jnp.int32),
            pltpu.VMEM((CH, CHW), jnp.int32),
            pltpu.VMEM((CH, CHW), jnp.int32),
            pltpu.VMEM((CH, CHW), jnp.int32),
            pltpu.VMEM((E, CH, CHW), jnp.float32),
            pltpu.VMEM((E, CH, CHW), jnp.float32),
            pltpu.VMEM((2 * E, CH, CHW), jnp.float32),
            pltpu.VMEM((2 * E, CH, CHW), jnp.float32),
            pltpu.SemaphoreType.DMA,
        ],
        compiler_params=pltpu.CompilerParams(use_tc_tiling_on_sc=False),
    )


WIN = 16


def _tc_body(emg_ref, etg_ref, mm_ref, tt_ref, w1a, w1b, b1r, w2, b2r, w3, b3r,
             wog, wom, bor, out_ref):
    dn = (((1,), (0,)), ((), ()))
    mm = mm_ref[:]
    tt = tt_ref[:]
    h = (lax.dot_general(w1a[:], mm, dn)
         + lax.dot_general(w1b[:], tt, dn) + b1r[:].reshape(64, 1, 1))
    h = jnp.maximum(h, 0.0)
    h = jnp.maximum(lax.dot_general(w2[:], h, dn) + b2r[:].reshape(32, 1, 1), 0.0)
    h = jnp.maximum(lax.dot_general(w3[:], h, dn) + b3r[:].reshape(16, 1, 1), 0.0)
    g = emg_ref[:] * etg_ref[:]
    out = (lax.dot_general(wog[:], g, dn)[0]
           + lax.dot_general(wom[:], h, dn)[0] + bor[0, 0])
    out_ref[:] = out


def _tc_mlp(emgT, etgT, mmT, ttT, w1a, w1b, b1r, w2, b2r, w3, b3r, wog, wom, bor):
    full2 = lambda shape: pl.BlockSpec(shape, lambda i: (0, 0))
    return pl.pallas_call(
        _tc_body,
        grid=(B // 128 // WIN,),
        in_specs=[
            pl.BlockSpec((E, WIN, 128), lambda i: (0, i, 0)),
            pl.BlockSpec((E, WIN, 128), lambda i: (0, i, 0)),
            pl.BlockSpec((2 * E, WIN, 128), lambda i: (0, i, 0)),
            pl.BlockSpec((2 * E, WIN, 128), lambda i: (0, i, 0)),
            full2((64, 2 * E)),
            full2((64, 2 * E)),
            pl.BlockSpec((64,), lambda i: (0,)),
            full2((32, 64)),
            pl.BlockSpec((32,), lambda i: (0,)),
            full2((16, 32)),
            pl.BlockSpec((16,), lambda i: (0,)),
            full2((1, E)),
            full2((1, 16)),
            full2((1, 1)),
        ],
        out_specs=pl.BlockSpec((WIN, 128), lambda i: (i, 0)),
        out_shape=jax.ShapeDtypeStruct((B // 128, 128), jnp.float32),
    )(emgT, etgT, mmT, ttT, w1a, w1b, b1r, w2, b2r, w3, b3r, wog, wom, bor)


def kernel(model_ids, task_ids, Emg, Etg, Emm, Etm, W1, b1, W2, b2, W3, b3, Wo, bo):
    mid2 = model_ids.reshape(B // CHW, CHW)
    tid2 = task_ids.reshape(B // CHW, CHW)
    ag = _tc_reblock(Emg.T, GM)
    tg = _tc_reblock(Etg.T, GT)
    am = _tc_reblock(Emm.T, GM)
    tm = _tc_reblock(Etm.T, GT)
    emgT, etgT, mmT, ttT = _sc_gather()(
        mid2, tid2,
        ag.reshape(-1), tg.reshape(-1), am.reshape(-1), tm.reshape(-1))
    out2 = _tc_mlp(
        emgT, etgT, mmT, ttT,
        W1[: 2 * E].T, W1[2 * E:].T, b1,
        W2.T, b2,
        W3.T, b3,
        Wo[:E].reshape(1, E), Wo[E:].reshape(1, 16),
        bo.reshape(1, 1),
    )
    return out2.reshape(B)


# speedup vs baseline: 18.9552x; 2.1225x over previous
import functools

import jax
import jax.numpy as jnp
from jax import lax
from jax.experimental import pallas as pl
from jax.experimental.pallas import tpu as pltpu
from jax.experimental.pallas import tpu_sc as plsc

B = 16384
E = 16
NM = 1000000
NT = 100000
NC = 2
NS = 16
NW = NC * NS
BPW = B // NW
CHW = 128
CH = BPW // CHW

GM = (NM + 127) // 128
GT = (NT + 127) // 128
MTAIL = NM - (GM - 1) * 128
TTAIL = NT - (GT - 1) * 128


RBK = 64


def _tc_reblock(tblT, n_groups):
    c_dim = tblT.shape[0]

    def body(in_ref, out_ref):
        x = in_ref[:]
        out_ref[:] = jnp.transpose(x.reshape(c_dim, RBK, 128), (1, 0, 2))

    return pl.pallas_call(
        body,
        grid=((n_groups + RBK - 1) // RBK,),
        in_specs=[pl.BlockSpec((c_dim, RBK * 128), lambda i: (0, i))],
        out_specs=pl.BlockSpec((RBK, c_dim, 128), lambda i: (i, 0, 0)),
        out_shape=jax.ShapeDtypeStruct((n_groups, c_dim, 128), jnp.float32),
    )(tblT)


def _off_segments(idx_ref, off_ref, span):
    for ch in range(CH):
        for k in range(CHW // 16):
            sl = pl.ds(k * 16, 16)
            v = idx_ref[ch, sl]
            off_ref[ch, sl] = (
                lax.shift_right_logical(v, 7) * span
                + lax.bitwise_and(v, 127))


def _sc_gather_body(mid_hbm, tid_hbm, ag, tg, am, tm,
                    emg_o, etg_o, emm_o, etm_o,
                    idx_m, idx_t, off_mg, off_mm, off_tg, off_tm,
                    emg_v, etg_v, emm_v, etm_v, sem):
    wid = lax.axis_index("s") * NC + lax.axis_index("c")
    row0 = wid * CH
    pltpu.sync_copy(mid_hbm.at[pl.ds(row0, CH)], idx_m)
    pltpu.sync_copy(tid_hbm.at[pl.ds(row0, CH)], idx_t)
    _off_segments(idx_m, off_mg, E * 128)
    _off_segments(idx_m, off_mm, 2 * E * 128)
    _off_segments(idx_t, off_tg, E * 128)
    _off_segments(idx_t, off_tm, 2 * E * 128)

    copies = []
    for ch in range(CH):
        for c in range(E):
            copies.append(pltpu.async_copy(
                ag.at[pl.ds(c * 128, GM * E * 128 - c * 128)].at[off_mg.at[ch]],
                emg_v.at[c, ch], sem))
            copies.append(pltpu.async_copy(
                tg.at[pl.ds(c * 128, GT * E * 128 - c * 128)].at[off_tg.at[ch]],
                etg_v.at[c, ch], sem))
        for c in range(2 * E):
            copies.append(pltpu.async_copy(
                am.at[pl.ds(c * 128, GM * 2 * E * 128 - c * 128)].at[off_mm.at[ch]],
                emm_v.at[c, ch], sem))
            copies.append(pltpu.async_copy(
                tm.at[pl.ds(c * 128, GT * 2 * E * 128 - c * 128)].at[off_tm.at[ch]],
                etm_v.at[c, ch], sem))
    for cp in copies:
        cp.wait()

    w0 = wid * CH
    pltpu.sync_copy(emg_v, emg_o.at[:, pl.ds(w0, CH), :])
    pltpu.sync_copy(etg_v, etg_o.at[:, pl.ds(w0, CH), :])
    pltpu.sync_copy(emm_v, emm_o.at[:, pl.ds(w0, CH), :])
    pltpu.sync_copy(etm_v, etm_o.at[:, pl.ds(w0, CH), :])


@functools.cache
def _sc_gather():
    return pl.kernel(
        _sc_gather_body,
        out_type=(
            jax.ShapeDtypeStruct((E, B // 128, 128), jnp.float32),
            jax.ShapeDtypeStruct((E, B // 128, 128), jnp.float32),
            jax.ShapeDtypeStruct((2 * E, B // 128, 128), jnp.float32),
            jax.ShapeDtypeStruct((2 * E, B // 128, 128), jnp.float32),
        ),
        mesh=plsc.VectorSubcoreMesh(core_axis_name="c", subcore_axis_name="s"),
        scratch_types=[
            pltpu.VMEM((CH, CHW), jnp.int32),
            pltpu.VMEM((CH, CHW), jnp.int32),
            pltpu.VMEM((CH, CHW), jnp.int32),
            pltpu.VMEM((CH, CHW), jnp.int32),
            pltpu.VMEM((CH, CHW), jnp.int32),
            pltpu.VMEM((CH, CHW), jnp.int32),
            pltpu.VMEM((E, CH, CHW), jnp.float32),
            pltpu.VMEM((E, CH, CHW), jnp.float32),
            pltpu.VMEM((2 * E, CH, CHW), jnp.float32),
            pltpu.VMEM((2 * E, CH, CHW), jnp.float32),
            pltpu.SemaphoreType.DMA,
        ],
        compiler_params=pltpu.CompilerParams(use_tc_tiling_on_sc=False),
    )


WIN = 16


def _tc_body(emg_ref, etg_ref, mm_ref, tt_ref, w1a, w1b, b1r, w2, b2r, w3, b3r,
             wog, wom, bor, out_ref):
    dn = (((1,), (0,)), ((), ()))
    mm = mm_ref[:]
    tt = tt_ref[:]
    h = (lax.dot_general(w1a[:], mm, dn)
         + lax.dot_general(w1b[:], tt, dn) + b1r[:].reshape(64, 1, 1))
    h = jnp.maximum(h, 0.0)
    h = jnp.maximum(lax.dot_general(w2[:], h, dn) + b2r[:].reshape(32, 1, 1), 0.0)
    h = jnp.maximum(lax.dot_general(w3[:], h, dn) + b3r[:].reshape(16, 1, 1), 0.0)
    g = emg_ref[:] * etg_ref[:]
    out = (lax.dot_general(wog[:], g, dn)[0]
           + lax.dot_general(wom[:], h, dn)[0] + bor[0, 0])
    out_ref[:] = out


def _tc_mlp(emgT, etgT, mmT, ttT, w1a, w1b, b1r, w2, b2r, w3, b3r, wog, wom, bor):
    full2 = lambda shape: pl.BlockSpec(shape, lambda i: (0, 0))
    return pl.pallas_call(
        _tc_body,
        grid=(B // 128 // WIN,),
        in_specs=[
            pl.BlockSpec((E, WIN, 128), lambda i: (0, i, 0)),
            pl.BlockSpec((E, WIN, 128), lambda i: (0, i, 0)),
            pl.BlockSpec((2 * E, WIN, 128), lambda i: (0, i, 0)),
            pl.BlockSpec((2 * E, WIN, 128), lambda i: (0, i, 0)),
            full2((64, 2 * E)),
            full2((64, 2 * E)),
            pl.BlockSpec((64,), lambda i: (0,)),
            full2((32, 64)),
            pl.BlockSpec((32,), lambda i: (0,)),
            full2((16, 32)),
            pl.BlockSpec((16,), lambda i: (0,)),
            full2((1, E)),
            full2((1, 16)),
            full2((1, 1)),
        ],
        out_specs=pl.BlockSpec((WIN, 128), lambda i: (i, 0)),
        out_shape=jax.ShapeDtypeStruct((B // 128, 128), jnp.float32),
    )(emgT, etgT, mmT, ttT, w1a, w1b, b1r, w2, b2r, w3, b3r, wog, wom, bor)


def kernel(model_ids, task_ids, Emg, Etg, Emm, Etm, W1, b1, W2, b2, W3, b3, Wo, bo):
    mid2 = model_ids.reshape(B // CHW, CHW)
    tid2 = task_ids.reshape(B // CHW, CHW)
    ag = _tc_reblock(Emg.T, GM)
    tg = _tc_reblock(Etg.T, GT)
    am = _tc_reblock(Emm.T, GM)
    tm = _tc_reblock(Etm.T, GT)
    emgT, etgT, mmT, ttT = _sc_gather()(
        mid2, tid2,
        ag.reshape(-1), tg.reshape(-1), am.reshape(-1), tm.reshape(-1))
    out2 = _tc_mlp(
        emgT, etgT, mmT, ttT,
        W1[: 2 * E].T, W1[2 * E:].T, b1,
        W2.T, b2,
        W3.T, b3,
        Wo[:E].reshape(1, E), Wo[E:].reshape(1, 16),
        bo.reshape(1, 1),
    )
    return out2.reshape(B)


# speedup vs baseline: 23.5432x; 1.2420x over previous
import functools

import jax
import jax.numpy as jnp
from jax import lax
from jax.experimental import pallas as pl
from jax.experimental.pallas import tpu as pltpu
from jax.experimental.pallas import tpu_sc as plsc

B = 16384
E = 16
NM = 1000000
NT = 100000
NC = 2
NS = 16
NW = NC * NS
BPW = B // NW
CHW = 128
CH = BPW // CHW

GM = (NM + 127) // 128
GT = (NT + 127) // 128
MTAIL = NM - (GM - 1) * 128
TTAIL = NT - (GT - 1) * 128


RBK = 128


def _tc_reblock(tblT, n_groups):
    c_dim = tblT.shape[0]

    def body(in_ref, out_ref):
        x = in_ref[:]
        out_ref[:] = jnp.transpose(x.reshape(c_dim, RBK, 128), (1, 0, 2))

    return pl.pallas_call(
        body,
        grid=((n_groups + RBK - 1) // RBK,),
        in_specs=[pl.BlockSpec((c_dim, RBK * 128), lambda i: (0, i))],
        out_specs=pl.BlockSpec((RBK, c_dim, 128), lambda i: (i, 0, 0)),
        out_shape=jax.ShapeDtypeStruct((n_groups, c_dim, 128), jnp.float32),
    )(tblT)


def _off_segments(idx_ref, off_ref, span):
    for ch in range(CH):
        for k in range(CHW // 16):
            sl = pl.ds(k * 16, 16)
            v = idx_ref[ch, sl]
            off_ref[ch, sl] = (
                lax.shift_right_logical(v, 7) * span
                + lax.bitwise_and(v, 127))


def _sc_gather_body(mid_hbm, tid_hbm, ag, tg, am, tm,
                    emg_o, etg_o, emm_o, etm_o,
                    idx_m, idx_t, off_mg, off_mm, off_tg, off_tm,
                    emg_v, etg_v, emm_v, etm_v, sem):
    wid = lax.axis_index("s") * NC + lax.axis_index("c")
    row0 = wid * CH
    pltpu.sync_copy(mid_hbm.at[pl.ds(row0, CH)], idx_m)
    pltpu.sync_copy(tid_hbm.at[pl.ds(row0, CH)], idx_t)
    _off_segments(idx_m, off_mg, E * 128)
    _off_segments(idx_m, off_mm, 2 * E * 128)
    _off_segments(idx_t, off_tg, E * 128)
    _off_segments(idx_t, off_tm, 2 * E * 128)

    copies = []
    for ch in range(CH):
        for c in range(E):
            copies.append(pltpu.async_copy(
                ag.at[pl.ds(c * 128, GM * E * 128 - c * 128)].at[off_mg.at[ch]],
                emg_v.at[c, ch], sem))
            copies.append(pltpu.async_copy(
                tg.at[pl.ds(c * 128, GT * E * 128 - c * 128)].at[off_tg.at[ch]],
                etg_v.at[c, ch], sem))
        for c in range(2 * E):
            copies.append(pltpu.async_copy(
                am.at[pl.ds(c * 128, GM * 2 * E * 128 - c * 128)].at[off_mm.at[ch]],
                emm_v.at[c, ch], sem))
            copies.append(pltpu.async_copy(
                tm.at[pl.ds(c * 128, GT * 2 * E * 128 - c * 128)].at[off_tm.at[ch]],
                etm_v.at[c, ch], sem))
    for cp in copies:
        cp.wait()

    w0 = wid * CH
    pltpu.sync_copy(emg_v, emg_o.at[:, pl.ds(w0, CH), :])
    pltpu.sync_copy(etg_v, etg_o.at[:, pl.ds(w0, CH), :])
    pltpu.sync_copy(emm_v, emm_o.at[:, pl.ds(w0, CH), :])
    pltpu.sync_copy(etm_v, etm_o.at[:, pl.ds(w0, CH), :])


@functools.cache
def _sc_gather():
    return pl.kernel(
        _sc_gather_body,
        out_type=(
            jax.ShapeDtypeStruct((E, B // 128, 128), jnp.float32),
            jax.ShapeDtypeStruct((E, B // 128, 128), jnp.float32),
            jax.ShapeDtypeStruct((2 * E, B // 128, 128), jnp.float32),
            jax.ShapeDtypeStruct((2 * E, B // 128, 128), jnp.float32),
        ),
        mesh=plsc.VectorSubcoreMesh(core_axis_name="c", subcore_axis_name="s"),
        scratch_types=[
            pltpu.VMEM((CH, CHW), jnp.int32),
            pltpu.VMEM((CH, CHW), jnp.int32),
            pltpu.VMEM((CH, CHW), jnp.int32),
            pltpu.VMEM((CH, CHW), jnp.int32),
            pltpu.VMEM((CH, CHW), jnp.int32),
            pltpu.VMEM((CH, CHW), jnp.int32),
            pltpu.VMEM((E, CH, CHW), jnp.float32),
            pltpu.VMEM((E, CH, CHW), jnp.float32),
            pltpu.VMEM((2 * E, CH, CHW), jnp.float32),
            pltpu.VMEM((2 * E, CH, CHW), jnp.float32),
            pltpu.SemaphoreType.DMA,
        ],
        compiler_params=pltpu.CompilerParams(use_tc_tiling_on_sc=False),
    )


WIN = 16


def _tc_body(emg_ref, etg_ref, mm_ref, tt_ref, w1a, w1b, b1r, w2, b2r, w3, b3r,
             wog, wom, bor, out_ref):
    dn = (((1,), (0,)), ((), ()))
    mm = mm_ref[:]
    tt = tt_ref[:]
    h = (lax.dot_general(w1a[:], mm, dn)
         + lax.dot_general(w1b[:], tt, dn) + b1r[:].reshape(64, 1, 1))
    h = jnp.maximum(h, 0.0)
    h = jnp.maximum(lax.dot_general(w2[:], h, dn) + b2r[:].reshape(32, 1, 1), 0.0)
    h = jnp.maximum(lax.dot_general(w3[:], h, dn) + b3r[:].reshape(16, 1, 1), 0.0)
    g = emg_ref[:] * etg_ref[:]
    out = (lax.dot_general(wog[:], g, dn)[0]
           + lax.dot_general(wom[:], h, dn)[0] + bor[0, 0])
    out_ref[:] = out


def _tc_mlp(emgT, etgT, mmT, ttT, w1a, w1b, b1r, w2, b2r, w3, b3r, wog, wom, bor):
    full2 = lambda shape: pl.BlockSpec(shape, lambda i: (0, 0))
    return pl.pallas_call(
        _tc_body,
        grid=(B // 128 // WIN,),
        in_specs=[
            pl.BlockSpec((E, WIN, 128), lambda i: (0, i, 0)),
            pl.BlockSpec((E, WIN, 128), lambda i: (0, i, 0)),
            pl.BlockSpec((2 * E, WIN, 128), lambda i: (0, i, 0)),
            pl.BlockSpec((2 * E, WIN, 128), lambda i: (0, i, 0)),
            full2((64, 2 * E)),
            full2((64, 2 * E)),
            pl.BlockSpec((64,), lambda i: (0,)),
            full2((32, 64)),
            pl.BlockSpec((32,), lambda i: (0,)),
            full2((16, 32)),
            pl.BlockSpec((16,), lambda i: (0,)),
            full2((1, E)),
            full2((1, 16)),
            full2((1, 1)),
        ],
        out_specs=pl.BlockSpec((WIN, 128), lambda i: (i, 0)),
        out_shape=jax.ShapeDtypeStruct((B // 128, 128), jnp.float32),
    )(emgT, etgT, mmT, ttT, w1a, w1b, b1r, w2, b2r, w3, b3r, wog, wom, bor)


def kernel(model_ids, task_ids, Emg, Etg, Emm, Etm, W1, b1, W2, b2, W3, b3, Wo, bo):
    mid2 = model_ids.reshape(B // CHW, CHW)
    tid2 = task_ids.reshape(B // CHW, CHW)
    ag = _tc_reblock(Emg.T, GM)
    tg = _tc_reblock(Etg.T, GT)
    am = _tc_reblock(Emm.T, GM)
    tm = _tc_reblock(Etm.T, GT)
    emgT, etgT, mmT, ttT = _sc_gather()(
        mid2, tid2,
        ag.reshape(-1), tg.reshape(-1), am.reshape(-1), tm.reshape(-1))
    out2 = _tc_mlp(
        emgT, etgT, mmT, ttT,
        W1[: 2 * E].T, W1[2 * E:].T, b1,
        W2.T, b2,
        W3.T, b3,
        Wo[:E].reshape(1, E), Wo[E:].reshape(1, 16),
        bo.reshape(1, 1),
    )
    return out2.reshape(B)


# speedup vs baseline: 26.9414x; 1.1443x over previous
import functools

import jax
import jax.numpy as jnp
from jax import lax
from jax.experimental import pallas as pl
from jax.experimental.pallas import tpu as pltpu
from jax.experimental.pallas import tpu_sc as plsc

B = 16384
E = 16
NM = 1000000
NT = 100000
NC = 2
NS = 16
NW = NC * NS
BPW = B // NW
CHW = 128
CH = BPW // CHW

GM = (NM + 127) // 128
GT = (NT + 127) // 128
MTAIL = NM - (GM - 1) * 128
TTAIL = NT - (GT - 1) * 128


RBK = 256


def _tc_reblock(tblT, n_groups):
    c_dim = tblT.shape[0]

    def body(in_ref, out_ref):
        x = in_ref[:]
        out_ref[:] = jnp.transpose(x.reshape(c_dim, RBK, 128), (1, 0, 2))

    return pl.pallas_call(
        body,
        grid=((n_groups + RBK - 1) // RBK,),
        in_specs=[pl.BlockSpec((c_dim, RBK * 128), lambda i: (0, i))],
        out_specs=pl.BlockSpec((RBK, c_dim, 128), lambda i: (i, 0, 0)),
        out_shape=jax.ShapeDtypeStruct((n_groups, c_dim, 128), jnp.float32),
    )(tblT)


def _off_segments(idx_ref, off_ref, span):
    for ch in range(CH):
        for k in range(CHW // 16):
            sl = pl.ds(k * 16, 16)
            v = idx_ref[ch, sl]
            off_ref[ch, sl] = (
                lax.shift_right_logical(v, 7) * span
                + lax.bitwise_and(v, 127))


def _sc_gather_body(mid_hbm, tid_hbm, ag, tg, am, tm,
                    emg_o, etg_o, emm_o, etm_o,
                    idx_m, idx_t, off_mg, off_mm, off_tg, off_tm,
                    emg_v, etg_v, emm_v, etm_v, sem):
    wid = lax.axis_index("s") * NC + lax.axis_index("c")
    row0 = wid * CH
    pltpu.sync_copy(mid_hbm.at[pl.ds(row0, CH)], idx_m)
    pltpu.sync_copy(tid_hbm.at[pl.ds(row0, CH)], idx_t)
    _off_segments(idx_m, off_mg, E * 128)
    _off_segments(idx_m, off_mm, 2 * E * 128)
    _off_segments(idx_t, off_tg, E * 128)
    _off_segments(idx_t, off_tm, 2 * E * 128)

    copies = []
    for ch in range(CH):
        for c in range(E):
            copies.append(pltpu.async_copy(
                ag.at[pl.ds(c * 128, GM * E * 128 - c * 128)].at[off_mg.at[ch]],
                emg_v.at[c, ch], sem))
            copies.append(pltpu.async_copy(
                tg.at[pl.ds(c * 128, GT * E * 128 - c * 128)].at[off_tg.at[ch]],
                etg_v.at[c, ch], sem))
        for c in range(2 * E):
            copies.append(pltpu.async_copy(
                am.at[pl.ds(c * 128, GM * 2 * E * 128 - c * 128)].at[off_mm.at[ch]],
                emm_v.at[c, ch], sem))
            copies.append(pltpu.async_copy(
                tm.at[pl.ds(c * 128, GT * 2 * E * 128 - c * 128)].at[off_tm.at[ch]],
                etm_v.at[c, ch], sem))
    for cp in copies:
        cp.wait()

    w0 = wid * CH
    pltpu.sync_copy(emg_v, emg_o.at[:, pl.ds(w0, CH), :])
    pltpu.sync_copy(etg_v, etg_o.at[:, pl.ds(w0, CH), :])
    pltpu.sync_copy(emm_v, emm_o.at[:, pl.ds(w0, CH), :])
    pltpu.sync_copy(etm_v, etm_o.at[:, pl.ds(w0, CH), :])


@functools.cache
def _sc_gather():
    return pl.kernel(
        _sc_gather_body,
        out_type=(
            jax.ShapeDtypeStruct((E, B // 128, 128), jnp.float32),
            jax.ShapeDtypeStruct((E, B // 128, 128), jnp.float32),
            jax.ShapeDtypeStruct((2 * E, B // 128, 128), jnp.float32),
            jax.ShapeDtypeStruct((2 * E, B // 128, 128), jnp.float32),
        ),
        mesh=plsc.VectorSubcoreMesh(core_axis_name="c", subcore_axis_name="s"),
        scratch_types=[
            pltpu.VMEM((CH, CHW), jnp.int32),
            pltpu.VMEM((CH, CHW), jnp.int32),
            pltpu.VMEM((CH, CHW), jnp.int32),
            pltpu.VMEM((CH, CHW), jnp.int32),
            pltpu.VMEM((CH, CHW), jnp.int32),
            pltpu.VMEM((CH, CHW), jnp.int32),
            pltpu.VMEM((E, CH, CHW), jnp.float32),
            pltpu.VMEM((E, CH, CHW), jnp.float32),
            pltpu.VMEM((2 * E, CH, CHW), jnp.float32),
            pltpu.VMEM((2 * E, CH, CHW), jnp.float32),
            pltpu.SemaphoreType.DMA,
        ],
        compiler_params=pltpu.CompilerParams(use_tc_tiling_on_sc=False),
    )


WIN = 16


def _tc_body(emg_ref, etg_ref, mm_ref, tt_ref, w1a, w1b, b1r, w2, b2r, w3, b3r,
             wog, wom, bor, out_ref):
    dn = (((1,), (0,)), ((), ()))
    mm = mm_ref[:]
    tt = tt_ref[:]
    h = (lax.dot_general(w1a[:], mm, dn)
         + lax.dot_general(w1b[:], tt, dn) + b1r[:].reshape(64, 1, 1))
    h = jnp.maximum(h, 0.0)
    h = jnp.maximum(lax.dot_general(w2[:], h, dn) + b2r[:].reshape(32, 1, 1), 0.0)
    h = jnp.maximum(lax.dot_general(w3[:], h, dn) + b3r[:].reshape(16, 1, 1), 0.0)
    g = emg_ref[:] * etg_ref[:]
    out = (lax.dot_general(wog[:], g, dn)[0]
           + lax.dot_general(wom[:], h, dn)[0] + bor[0, 0])
    out_ref[:] = out


def _tc_mlp(emgT, etgT, mmT, ttT, w1a, w1b, b1r, w2, b2r, w3, b3r, wog, wom, bor):
    full2 = lambda shape: pl.BlockSpec(shape, lambda i: (0, 0))
    return pl.pallas_call(
        _tc_body,
        grid=(B // 128 // WIN,),
        in_specs=[
            pl.BlockSpec((E, WIN, 128), lambda i: (0, i, 0)),
            pl.BlockSpec((E, WIN, 128), lambda i: (0, i, 0)),
            pl.BlockSpec((2 * E, WIN, 128), lambda i: (0, i, 0)),
            pl.BlockSpec((2 * E, WIN, 128), lambda i: (0, i, 0)),
            full2((64, 2 * E)),
            full2((64, 2 * E)),
            pl.BlockSpec((64,), lambda i: (0,)),
            full2((32, 64)),
            pl.BlockSpec((32,), lambda i: (0,)),
            full2((16, 32)),
            pl.BlockSpec((16,), lambda i: (0,)),
            full2((1, E)),
            full2((1, 16)),
            full2((1, 1)),
        ],
        out_specs=pl.BlockSpec((WIN, 128), lambda i: (i, 0)),
        out_shape=jax.ShapeDtypeStruct((B // 128, 128), jnp.float32),
    )(emgT, etgT, mmT, ttT, w1a, w1b, b1r, w2, b2r, w3, b3r, wog, wom, bor)


def kernel(model_ids, task_ids, Emg, Etg, Emm, Etm, W1, b1, W2, b2, W3, b3, Wo, bo):
    mid2 = model_ids.reshape(B // CHW, CHW)
    tid2 = task_ids.reshape(B // CHW, CHW)
    ag = _tc_reblock(Emg.T, GM)
    tg = _tc_reblock(Etg.T, GT)
    am = _tc_reblock(Emm.T, GM)
    tm = _tc_reblock(Etm.T, GT)
    emgT, etgT, mmT, ttT = _sc_gather()(
        mid2, tid2,
        ag.reshape(-1), tg.reshape(-1), am.reshape(-1), tm.reshape(-1))
    out2 = _tc_mlp(
        emgT, etgT, mmT, ttT,
        W1[: 2 * E].T, W1[2 * E:].T, b1,
        W2.T, b2,
        W3.T, b3,
        Wo[:E].reshape(1, E), Wo[E:].reshape(1, 16),
        bo.reshape(1, 1),
    )
    return out2.reshape(B)


# speedup vs baseline: 28.1223x; 1.0438x over previous
import functools

import jax
import jax.numpy as jnp
from jax import lax
from jax.experimental import pallas as pl
from jax.experimental.pallas import tpu as pltpu
from jax.experimental.pallas import tpu_sc as plsc

B = 16384
E = 16
NM = 1000000
NT = 100000
NC = 2
NS = 16
NW = NC * NS
BPW = B // NW
CHW = 128
CH = BPW // CHW

GM = (NM + 127) // 128
GT = (NT + 127) // 128
MTAIL = NM - (GM - 1) * 128
TTAIL = NT - (GT - 1) * 128


RBK = 512


def _tc_reblock(tblT, n_groups):
    c_dim = tblT.shape[0]

    def body(in_ref, out_ref):
        x = in_ref[:]
        out_ref[:] = jnp.transpose(x.reshape(c_dim, RBK, 128), (1, 0, 2))

    return pl.pallas_call(
        body,
        grid=((n_groups + RBK - 1) // RBK,),
        in_specs=[pl.BlockSpec((c_dim, RBK * 128), lambda i: (0, i))],
        out_specs=pl.BlockSpec((RBK, c_dim, 128), lambda i: (i, 0, 0)),
        out_shape=jax.ShapeDtypeStruct((n_groups, c_dim, 128), jnp.float32),
    )(tblT)


def _off_segments(idx_ref, off_ref, span):
    for ch in range(CH):
        for k in range(CHW // 16):
            sl = pl.ds(k * 16, 16)
            v = idx_ref[ch, sl]
            off_ref[ch, sl] = (
                lax.shift_right_logical(v, 7) * span
                + lax.bitwise_and(v, 127))


def _sc_gather_body(mid_hbm, tid_hbm, ag, tg, am, tm,
                    emg_o, etg_o, emm_o, etm_o,
                    idx_m, idx_t, off_mg, off_mm, off_tg, off_tm,
                    emg_v, etg_v, emm_v, etm_v, sem):
    wid = lax.axis_index("s") * NC + lax.axis_index("c")
    row0 = wid * CH
    pltpu.sync_copy(mid_hbm.at[pl.ds(row0, CH)], idx_m)
    pltpu.sync_copy(tid_hbm.at[pl.ds(row0, CH)], idx_t)
    _off_segments(idx_m, off_mg, E * 128)
    _off_segments(idx_m, off_mm, 2 * E * 128)
    _off_segments(idx_t, off_tg, E * 128)
    _off_segments(idx_t, off_tm, 2 * E * 128)

    copies = []
    for ch in range(CH):
        for c in range(E):
            copies.append(pltpu.async_copy(
                ag.at[pl.ds(c * 128, GM * E * 128 - c * 128)].at[off_mg.at[ch]],
                emg_v.at[c, ch], sem))
            copies.append(pltpu.async_copy(
                tg.at[pl.ds(c * 128, GT * E * 128 - c * 128)].at[off_tg.at[ch]],
                etg_v.at[c, ch], sem))
        for c in range(2 * E):
            copies.append(pltpu.async_copy(
                am.at[pl.ds(c * 128, GM * 2 * E * 128 - c * 128)].at[off_mm.at[ch]],
                emm_v.at[c, ch], sem))
            copies.append(pltpu.async_copy(
                tm.at[pl.ds(c * 128, GT * 2 * E * 128 - c * 128)].at[off_tm.at[ch]],
                etm_v.at[c, ch], sem))
    for cp in copies:
        cp.wait()

    w0 = wid * CH
    pltpu.sync_copy(emg_v, emg_o.at[:, pl.ds(w0, CH), :])
    pltpu.sync_copy(etg_v, etg_o.at[:, pl.ds(w0, CH), :])
    pltpu.sync_copy(emm_v, emm_o.at[:, pl.ds(w0, CH), :])
    pltpu.sync_copy(etm_v, etm_o.at[:, pl.ds(w0, CH), :])


@functools.cache
def _sc_gather():
    return pl.kernel(
        _sc_gather_body,
        out_type=(
            jax.ShapeDtypeStruct((E, B // 128, 128), jnp.float32),
            jax.ShapeDtypeStruct((E, B // 128, 128), jnp.float32),
            jax.ShapeDtypeStruct((2 * E, B // 128, 128), jnp.float32),
            jax.ShapeDtypeStruct((2 * E, B // 128, 128), jnp.float32),
        ),
        mesh=plsc.VectorSubcoreMesh(core_axis_name="c", subcore_axis_name="s"),
        scratch_types=[
            pltpu.VMEM((CH, CHW), jnp.int32),
            pltpu.VMEM((CH, CHW), jnp.int32),
            pltpu.VMEM((CH, CHW), jnp.int32),
            pltpu.VMEM((CH, CHW), jnp.int32),
            pltpu.VMEM((CH, CHW), jnp.int32),
            pltpu.VMEM((CH, CHW), jnp.int32),
            pltpu.VMEM((E, CH, CHW), jnp.float32),
            pltpu.VMEM((E, CH, CHW), jnp.float32),
            pltpu.VMEM((2 * E, CH, CHW), jnp.float32),
            pltpu.VMEM((2 * E, CH, CHW), jnp.float32),
            pltpu.SemaphoreType.DMA,
        ],
        compiler_params=pltpu.CompilerParams(use_tc_tiling_on_sc=False),
    )


WIN = 16


def _tc_body(emg_ref, etg_ref, mm_ref, tt_ref, w1a, w1b, b1r, w2, b2r, w3, b3r,
             wog, wom, bor, out_ref):
    dn = (((1,), (0,)), ((), ()))
    mm = mm_ref[:]
    tt = tt_ref[:]
    h = (lax.dot_general(w1a[:], mm, dn)
         + lax.dot_general(w1b[:], tt, dn) + b1r[:].reshape(64, 1, 1))
    h = jnp.maximum(h, 0.0)
    h = jnp.maximum(lax.dot_general(w2[:], h, dn) + b2r[:].reshape(32, 1, 1), 0.0)
    h = jnp.maximum(lax.dot_general(w3[:], h, dn) + b3r[:].reshape(16, 1, 1), 0.0)
    g = emg_ref[:] * etg_ref[:]
    out = (lax.dot_general(wog[:], g, dn)[0]
           + lax.dot_general(wom[:], h, dn)[0] + bor[0, 0])
    out_ref[:] = out


def _tc_mlp(emgT, etgT, mmT, ttT, w1a, w1b, b1r, w2, b2r, w3, b3r, wog, wom, bor):
    full2 = lambda shape: pl.BlockSpec(shape, lambda i: (0, 0))
    return pl.pallas_call(
        _tc_body,
        grid=(B // 128 // WIN,),
        in_specs=[
            pl.BlockSpec((E, WIN, 128), lambda i: (0, i, 0)),
            pl.BlockSpec((E, WIN, 128), lambda i: (0, i, 0)),
            pl.BlockSpec((2 * E, WIN, 128), lambda i: (0, i, 0)),
            pl.BlockSpec((2 * E, WIN, 128), lambda i: (0, i, 0)),
            full2((64, 2 * E)),
            full2((64, 2 * E)),
            pl.BlockSpec((64,), lambda i: (0,)),
            full2((32, 64)),
            pl.BlockSpec((32,), lambda i: (0,)),
            full2((16, 32)),
            pl.BlockSpec((16,), lambda i: (0,)),
            full2((1, E)),
            full2((1, 16)),
            full2((1, 1)),
        ],
        out_specs=pl.BlockSpec((WIN, 128), lambda i: (i, 0)),
        out_shape=jax.ShapeDtypeStruct((B // 128, 128), jnp.float32),
    )(emgT, etgT, mmT, ttT, w1a, w1b, b1r, w2, b2r, w3, b3r, wog, wom, bor)


def kernel(model_ids, task_ids, Emg, Etg, Emm, Etm, W1, b1, W2, b2, W3, b3, Wo, bo):
    mid2 = model_ids.reshape(B // CHW, CHW)
    tid2 = task_ids.reshape(B // CHW, CHW)
    ag = _tc_reblock(Emg.T, GM)
    tg = _tc_reblock(Etg.T, GT)
    am = _tc_reblock(Emm.T, GM)
    tm = _tc_reblock(Etm.T, GT)
    emgT, etgT, mmT, ttT = _sc_gather()(
        mid2, tid2,
        ag.reshape(-1), tg.reshape(-1), am.reshape(-1), tm.reshape(-1))
    out2 = _tc_mlp(
        emgT, etgT, mmT, ttT,
        W1[: 2 * E].T, W1[2 * E:].T, b1,
        W2.T, b2,
        W3.T, b3,
        Wo[:E].reshape(1, E), Wo[E:].reshape(1, 16),
        bo.reshape(1, 1),
    )
    return out2.reshape(B)


# speedup vs baseline: 28.1562x; 1.0012x over previous
import functools

import jax
import jax.numpy as jnp
from jax import lax
from jax.experimental import pallas as pl
from jax.experimental.pallas import tpu as pltpu
from jax.experimental.pallas import tpu_sc as plsc

B = 16384
E = 16
NM = 1000000
NT = 100000
NC = 2
NS = 16
NW = NC * NS
BPW = B // NW
CHW = 128
CH = BPW // CHW

GM = (NM + 127) // 128
GT = (NT + 127) // 128


RBK = 512


def _tc_reblock(tblT, n_groups):
    c_dim = tblT.shape[0]

    def body(in_ref, out_ref):
        x = in_ref[:]
        out_ref[:] = jnp.transpose(x.reshape(c_dim, RBK, 128), (1, 0, 2))

    return pl.pallas_call(
        body,
        grid=((n_groups + RBK - 1) // RBK,),
        in_specs=[pl.BlockSpec((c_dim, RBK * 128), lambda i: (0, i))],
        out_specs=pl.BlockSpec((RBK, c_dim, 128), lambda i: (i, 0, 0)),
        out_shape=jax.ShapeDtypeStruct((n_groups, c_dim, 128), jnp.float32),
    )(tblT)


def _off_segments(idx_ref, off_ref, span):
    for ch in range(CH):
        for k in range(CHW // 16):
            sl = pl.ds(k * 16, 16)
            v = idx_ref[ch, sl]
            off_ref[ch, sl] = (
                lax.shift_right_logical(v, 7) * span
                + lax.bitwise_and(v, 127))


def _sc_gather_body(mid_hbm, tid_hbm, ag, tg, am, tm,
                    emg_o, etg_o, emm_o, etm_o,
                    idx_m, idx_t, off_mg, off_mm, off_tg, off_tm,
                    emg_v, etg_v, emm_v, etm_v, sem):
    wid = lax.axis_index("s") * NC + lax.axis_index("c")
    row0 = wid * CH
    pltpu.sync_copy(mid_hbm.at[pl.ds(row0, CH)], idx_m)
    pltpu.sync_copy(tid_hbm.at[pl.ds(row0, CH)], idx_t)
    _off_segments(idx_m, off_mg, E * 128)
    _off_segments(idx_m, off_mm, 2 * E * 128)
    _off_segments(idx_t, off_tg, E * 128)
    _off_segments(idx_t, off_tm, 2 * E * 128)

    copies = []
    for ch in range(CH):
        for c in range(E):
            copies.append(pltpu.async_copy(
                ag.at[pl.ds(c * 128, GM * E * 128 - c * 128)].at[off_mg.at[ch]],
                emg_v.at[c, ch], sem))
            copies.append(pltpu.async_copy(
                tg.at[pl.ds(c * 128, GT * E * 128 - c * 128)].at[off_tg.at[ch]],
                etg_v.at[c, ch], sem))
        for c in range(2 * E):
            copies.append(pltpu.async_copy(
                am.at[pl.ds(c * 128, GM * 2 * E * 128 - c * 128)].at[off_mm.at[ch]],
                emm_v.at[c, ch], sem))
            copies.append(pltpu.async_copy(
                tm.at[pl.ds(c * 128, GT * 2 * E * 128 - c * 128)].at[off_tm.at[ch]],
                etm_v.at[c, ch], sem))
    for cp in copies:
        cp.wait()

    w0 = wid * CH
    pltpu.sync_copy(emg_v, emg_o.at[:, pl.ds(w0, CH), :])
    pltpu.sync_copy(etg_v, etg_o.at[:, pl.ds(w0, CH), :])
    pltpu.sync_copy(emm_v, emm_o.at[:, pl.ds(w0, CH), :])
    pltpu.sync_copy(etm_v, etm_o.at[:, pl.ds(w0, CH), :])


@functools.cache
def _sc_gather():
    return pl.kernel(
        _sc_gather_body,
        out_type=(
            jax.ShapeDtypeStruct((E, B // 128, 128), jnp.float32),
            jax.ShapeDtypeStruct((E, B // 128, 128), jnp.float32),
            jax.ShapeDtypeStruct((2 * E, B // 128, 128), jnp.float32),
            jax.ShapeDtypeStruct((2 * E, B // 128, 128), jnp.float32),
        ),
        mesh=plsc.VectorSubcoreMesh(core_axis_name="c", subcore_axis_name="s"),
        scratch_types=[
            pltpu.VMEM((CH, CHW), jnp.int32),
            pltpu.VMEM((CH, CHW), jnp.int32),
            pltpu.VMEM((CH, CHW), jnp.int32),
            pltpu.VMEM((CH, CHW), jnp.int32),
            pltpu.VMEM((CH, CHW), jnp.int32),
            pltpu.VMEM((CH, CHW), jnp.int32),
            pltpu.VMEM((E, CH, CHW), jnp.float32),
            pltpu.VMEM((E, CH, CHW), jnp.float32),
            pltpu.VMEM((2 * E, CH, CHW), jnp.float32),
            pltpu.VMEM((2 * E, CH, CHW), jnp.float32),
            pltpu.SemaphoreType.DMA,
        ],
        compiler_params=pltpu.CompilerParams(use_tc_tiling_on_sc=False),
    )


WIN = 16


def _tc_body(emg_ref, etg_ref, mm_ref, tt_ref, w1a, w1b, b1r, w2, b2r, w3, b3r,
             wog, wom, bor, out_ref):
    dn = (((1,), (0,)), ((), ()))
    mm = mm_ref[:]
    tt = tt_ref[:]
    h = (lax.dot_general(w1a[:], mm, dn)
         + lax.dot_general(w1b[:], tt, dn) + b1r[:].reshape(64, 1, 1))
    h = jnp.maximum(h, 0.0)
    h = jnp.maximum(lax.dot_general(w2[:], h, dn) + b2r[:].reshape(32, 1, 1), 0.0)
    h = jnp.maximum(lax.dot_general(w3[:], h, dn) + b3r[:].reshape(16, 1, 1), 0.0)
    g = emg_ref[:] * etg_ref[:]
    out = (lax.dot_general(wog[:], g, dn)[0]
           + lax.dot_general(wom[:], h, dn)[0] + bor[0, 0])
    out_ref[:] = out


def _tc_mlp(emgT, etgT, mmT, ttT, w1a, w1b, b1r, w2, b2r, w3, b3r, wog, wom, bor):
    full2 = lambda shape: pl.BlockSpec(shape, lambda i: (0, 0))
    return pl.pallas_call(
        _tc_body,
        grid=(B // 128 // WIN,),
        in_specs=[
            pl.BlockSpec((E, WIN, 128), lambda i: (0, i, 0)),
            pl.BlockSpec((E, WIN, 128), lambda i: (0, i, 0)),
            pl.BlockSpec((2 * E, WIN, 128), lambda i: (0, i, 0)),
            pl.BlockSpec((2 * E, WIN, 128), lambda i: (0, i, 0)),
            full2((64, 2 * E)),
            full2((64, 2 * E)),
            pl.BlockSpec((64,), lambda i: (0,)),
            full2((32, 64)),
            pl.BlockSpec((32,), lambda i: (0,)),
            full2((16, 32)),
            pl.BlockSpec((16,), lambda i: (0,)),
            full2((1, E)),
            full2((1, 16)),
            full2((1, 1)),
        ],
        out_specs=pl.BlockSpec((WIN, 128), lambda i: (i, 0)),
        out_shape=jax.ShapeDtypeStruct((B // 128, 128), jnp.float32),
    )(emgT, etgT, mmT, ttT, w1a, w1b, b1r, w2, b2r, w3, b3r, wog, wom, bor)


def kernel(model_ids, task_ids, Emg, Etg, Emm, Etm, W1, b1, W2, b2, W3, b3, Wo, bo):
    mid2 = model_ids.reshape(B // CHW, CHW)
    tid2 = task_ids.reshape(B // CHW, CHW)
    ag = _tc_reblock(Emg.T, GM)
    tg = _tc_reblock(Etg.T, GT)
    am = _tc_reblock(Emm.T, GM)
    tm = _tc_reblock(Etm.T, GT)
    emgT, etgT, mmT, ttT = _sc_gather()(
        mid2, tid2,
        ag.reshape(-1), tg.reshape(-1), am.reshape(-1), tm.reshape(-1))
    out2 = _tc_mlp(
        emgT, etgT, mmT, ttT,
        W1[: 2 * E].T, W1[2 * E:].T, b1,
        W2.T, b2,
        W3.T, b3,
        Wo[:E].reshape(1, E), Wo[E:].reshape(1, 16),
        bo.reshape(1, 1),
    )
    return out2.reshape(B)
